# Initial kernel scaffold; baseline (speedup 1.0000x reference)
import jax
import jax.numpy as jnp
from jax.experimental import pallas as pl


def kernel(x, edge_index, edge_attr, W1, as1, ad1, ae1, We1, b1, W2, as2, ad2, ae2, We2, b2, Wmu, bmu, Wlv, blv, Wd, bd, Wa1, ba1, Wa2, ba2, Wn1, bn1, Wn2, bn2):
    raise NotImplementedError("write your pallas kernel here")


# speedup vs baseline: 25.4013x; 25.4013x over previous
import functools

import jax
import jax.numpy as jnp
from jax import lax
from jax.experimental import pallas as pl
from jax.experimental.pallas import tpu as pltpu
from jax.experimental.pallas import tpu_sc as plsc

_N = 10000
_E = 320000
_DF = 128
_DE = 4
_HID = 64
_LD = 32
_MAXN = 200

_NP = 10240
_NPT = _NP // 16
_L = 128
_NW = 32
_CROWS = 16
_NCH = 5
_RPW = _CROWS * _NCH
_RP = _NW * _RPW
_EP = _RP * _L


@functools.lru_cache(maxsize=None)
def _sc_mesh():
    return plsc.VectorSubcoreMesh(core_axis_name="c", subcore_axis_name="s")


def _leaky(v):
    return jnp.maximum(v, 0.2 * v)


def _edge_logit_vreg(als_v, ald_v, ea_c, src_c, dst_c, j, l):
    sl = pl.ds(l * 16, 16)
    s16 = src_c[j, sl]
    d16 = dst_c[j, sl]
    a = (plsc.load_gather(als_v, [s16]) + plsc.load_gather(ald_v, [d16])
         + ea_c[j, sl])
    return _leaky(a), d16


def _scatter_max_vreg(amax_v, d16, a):
    def cond(done):
        return done == 0

    def body(done):
        cur = plsc.load_gather(amax_v, [d16])
        plsc.store_scatter(amax_v, [d16], a, mask=a > cur)
        chk = plsc.load_gather(amax_v, [d16])
        return lax.select(jnp.all(chk >= a), 1, 0)

    lax.while_loop(cond, body, 0)


def _zero_vec(ref, n):
    def zbody(i, _):
        ref[pl.ds(i * 16, 16)] = jnp.zeros((16,), jnp.float32)
        return 0
    lax.fori_loop(0, n // 16, zbody, 0)


def _sc_pass_a_body(with_esum, src_hbm, dst_hbm, ea_hbm, eat_hbm, als_hbm,
                    ald_hbm, amax_out, deg_out, esum_out,
                    als_v, ald_v, amax_v, src_c, dst_c, ea_c, eat_c, ones_v,
                    bnc_v, deg_sh, e0_sh, e1_sh, e2_sh, e3_sh):
    c = lax.axis_index("c")
    s = lax.axis_index("s")
    wid = s * 2 + c
    tsl = pl.ds(s * _NPT, _NPT)

    if with_esum:
        _zero_vec(bnc_v, _NPT)
        pltpu.sync_copy(bnc_v, deg_sh.at[tsl])
        pltpu.sync_copy(bnc_v, e0_sh.at[tsl])
        pltpu.sync_copy(bnc_v, e1_sh.at[tsl])
        pltpu.sync_copy(bnc_v, e2_sh.at[tsl])
        pltpu.sync_copy(bnc_v, e3_sh.at[tsl])

    pltpu.sync_copy(als_hbm, als_v)
    pltpu.sync_copy(ald_hbm, ald_v)

    def init_amax(i, _):
        amax_v[pl.ds(i * 16, 16)] = jnp.full((16,), -1e38, jnp.float32)
        return 0
    lax.fori_loop(0, _NP // 16, init_amax, 0)

    if with_esum:
        def init_ones(i, _):
            ones_v[pl.ds(i * 16, 16)] = jnp.full((16,), 1.0, jnp.float32)
            return 0
        lax.fori_loop(0, _L // 16, init_ones, 0)
        plsc.subcore_barrier()

    def do_row(j, _):
        if with_esum:
            idx = dst_c.at[j]
            pltpu.sync_copy(ones_v, deg_sh.at[idx], add=True)
            pltpu.sync_copy(eat_c.at[0, j], e0_sh.at[idx], add=True)
            pltpu.sync_copy(eat_c.at[1, j], e1_sh.at[idx], add=True)
            pltpu.sync_copy(eat_c.at[2, j], e2_sh.at[idx], add=True)
            pltpu.sync_copy(eat_c.at[3, j], e3_sh.at[idx], add=True)

        def vbody(l, _):
            a, d16 = _edge_logit_vreg(als_v, ald_v, ea_c, src_c, dst_c, j, l)
            _scatter_max_vreg(amax_v, d16, a)
            return 0
        lax.fori_loop(0, _L // 16, vbody, 0)
        return 0

    def do_chunk(ch, _):
        rowbase = wid * _RPW + ch * _CROWS
        rsl = pl.ds(rowbase, _CROWS)
        pltpu.sync_copy(src_hbm.at[rsl], src_c)
        pltpu.sync_copy(dst_hbm.at[rsl], dst_c)
        pltpu.sync_copy(ea_hbm.at[rsl], ea_c)
        if with_esum:
            for cc in range(4):
                pltpu.sync_copy(eat_hbm.at[cc, rsl], eat_c.at[cc])
        lax.fori_loop(0, _CROWS, do_row, 0)
        return 0

    lax.fori_loop(0, _NCH, do_chunk, 0)

    pltpu.sync_copy(amax_v, amax_out.at[pl.ds(wid * _NP, _NP)])

    if with_esum:
        plsc.subcore_barrier()
        base = c * _NP + s * _NPT
        osl = pl.ds(base, _NPT)
        pltpu.sync_copy(deg_sh.at[tsl], bnc_v)
        pltpu.sync_copy(bnc_v, deg_out.at[osl])
        for i, ref in enumerate((e0_sh, e1_sh, e2_sh, e3_sh)):
            pltpu.sync_copy(ref.at[tsl], bnc_v)
            pltpu.sync_copy(
                bnc_v, esum_out.at[pl.ds(c * 4 * _NP + i * _NP + s * _NPT,
                                         _NPT)])


@functools.lru_cache(maxsize=None)
def _make_sc_pass_a(with_esum):
    outs = [jax.ShapeDtypeStruct((_NW * _NP,), jnp.float32)]
    if with_esum:
        outs += [jax.ShapeDtypeStruct((2 * _NP,), jnp.float32),
                 jax.ShapeDtypeStruct((2 * 4 * _NP,), jnp.float32)]
    scratch = [
        pltpu.VMEM((_NP,), jnp.float32),
        pltpu.VMEM((_NP,), jnp.float32),
        pltpu.VMEM((_NP,), jnp.float32),
        pltpu.VMEM((_CROWS, _L), jnp.int32),
        pltpu.VMEM((_CROWS, _L), jnp.int32),
        pltpu.VMEM((_CROWS, _L), jnp.float32),
        pltpu.VMEM((4, _CROWS, _L), jnp.float32),
        pltpu.VMEM((_L,), jnp.float32),
        pltpu.VMEM((_NPT,), jnp.float32),
        pltpu.VMEM_SHARED((_NP,), jnp.float32),
        pltpu.VMEM_SHARED((_NP,), jnp.float32),
        pltpu.VMEM_SHARED((_NP,), jnp.float32),
        pltpu.VMEM_SHARED((_NP,), jnp.float32),
        pltpu.VMEM_SHARED((_NP,), jnp.float32),
    ]

    if with_esum:
        def body(src_hbm, dst_hbm, ea_hbm, eat_hbm, als_hbm, ald_hbm,
                 amax_out, deg_out, esum_out, *sc):
            _sc_pass_a_body(True, src_hbm, dst_hbm, ea_hbm, eat_hbm, als_hbm,
                            ald_hbm, amax_out, deg_out, esum_out, *sc)
    else:
        def body(src_hbm, dst_hbm, ea_hbm, als_hbm, ald_hbm, amax_out, *sc):
            _sc_pass_a_body(False, src_hbm, dst_hbm, ea_hbm, None, als_hbm,
                            ald_hbm, amax_out, None, None, *sc)

    return pl.kernel(body, mesh=_sc_mesh(), out_type=outs,
                     scratch_types=scratch,
                     compiler_params=pltpu.CompilerParams(
                         needs_layout_passes=False,
                         use_tc_tiling_on_sc=False))


def _sc_pass_b_body(src_hbm, dst_hbm, ea_hbm, als_hbm, ald_hbm, amax_hbm,
                    h_hbm, acc_out, den_out,
                    als_v, ald_v, amax_v, src_c, dst_c, ea_c, exs_c, rows_v,
                    bnc_v, acc_sh, den_sh, sem):
    c = lax.axis_index("c")
    s = lax.axis_index("s")
    wid = s * 2 + c
    tsl = pl.ds(s * _NPT, _NPT)

    _zero_vec(bnc_v, _NPT)
    pltpu.sync_copy(bnc_v, den_sh.at[tsl])

    def zrow(r, _):
        for cb in range(4):
            rows_v[r, pl.ds(cb * 16, 16)] = jnp.zeros((16,), jnp.float32)
        return 0
    lax.fori_loop(0, _L, zrow, 0)
    for k in range(_NPT // _L):
        pltpu.sync_copy(rows_v, acc_sh.at[pl.ds(s * _NPT + k * _L, _L)])

    pltpu.sync_copy(als_hbm, als_v)
    pltpu.sync_copy(ald_hbm, ald_v)
    pltpu.sync_copy(amax_hbm, amax_v)
    plsc.subcore_barrier()

    def do_row(j, _):
        pltpu.async_copy(h_hbm.at[src_c.at[j]], rows_v, sem).wait()

        def vbody(l, _):
            a, d16 = _edge_logit_vreg(als_v, ald_v, ea_c, src_c, dst_c, j, l)
            ex = jnp.exp(a - plsc.load_gather(amax_v, [d16]))
            exs_c[j, pl.ds(l * 16, 16)] = ex
            return 0
        lax.fori_loop(0, _L // 16, vbody, 0)

        def sbody(l, _):
            ex16 = exs_c[j, pl.ds(l * 16, 16)]
            for i in range(16):
                e = ex16[i]
                r = l * 16 + i
                for cb in range(4):
                    sl = pl.ds(cb * 16, 16)
                    rows_v[r, sl] = rows_v[r, sl] * e
            return 0
        lax.fori_loop(0, _L // 16, sbody, 0)

        idx = dst_c.at[j]
        pltpu.sync_copy(rows_v, acc_sh.at[idx], add=True)
        pltpu.sync_copy(exs_c.at[j], den_sh.at[idx], add=True)
        return 0

    def do_chunk(ch, _):
        rowbase = wid * _RPW + ch * _CROWS
        rsl = pl.ds(rowbase, _CROWS)
        pltpu.sync_copy(src_hbm.at[rsl], src_c)
        pltpu.sync_copy(dst_hbm.at[rsl], dst_c)
        pltpu.sync_copy(ea_hbm.at[rsl], ea_c)
        lax.fori_loop(0, _CROWS, do_row, 0)
        return 0

    lax.fori_loop(0, _NCH, do_chunk, 0)

    plsc.subcore_barrier()

    pltpu.sync_copy(den_sh.at[tsl], bnc_v)
    pltpu.sync_copy(bnc_v, den_out.at[pl.ds(c * _NP + s * _NPT, _NPT)])
    for k in range(_NPT // _L):
        ssl = pl.ds(s * _NPT + k * _L, _L)
        pltpu.sync_copy(acc_sh.at[ssl], rows_v)
        pltpu.sync_copy(rows_v,
                        acc_out.at[pl.ds(c * _NP + s * _NPT + k * _L, _L)])


@functools.lru_cache(maxsize=None)
def _make_sc_pass_b():
    return pl.kernel(
        _sc_pass_b_body,
        mesh=_sc_mesh(),
        out_type=[jax.ShapeDtypeStruct((2 * _NP, _HID), jnp.float32),
                  jax.ShapeDtypeStruct((2 * _NP,), jnp.float32)],
        scratch_types=[
            pltpu.VMEM((_NP,), jnp.float32),
            pltpu.VMEM((_NP,), jnp.float32),
            pltpu.VMEM((_NP,), jnp.float32),
            pltpu.VMEM((_CROWS, _L), jnp.int32),
            pltpu.VMEM((_CROWS, _L), jnp.int32),
            pltpu.VMEM((_CROWS, _L), jnp.float32),
            pltpu.VMEM((_CROWS, _L), jnp.float32),
            pltpu.VMEM((_L, _HID), jnp.float32),
            pltpu.VMEM((_NPT,), jnp.float32),
            pltpu.VMEM_SHARED((_NP, _HID), jnp.float32),
            pltpu.VMEM_SHARED((_NP,), jnp.float32),
            pltpu.SemaphoreType.DMA,
        ],
        compiler_params=pltpu.CompilerParams(needs_layout_passes=False,
                                             use_tc_tiling_on_sc=False),
    )


def _tc_prep_body(x_ref, w1_ref, as1_ref, ad1_ref, eat_ref, wv_ref,
                  h1_ref, als_ref, ald_ref, ea1_ref, ea2_ref):
    h = jnp.dot(x_ref[...], w1_ref[...], preferred_element_type=jnp.float32)
    h1_ref[...] = h
    als_ref[...] = jnp.dot(h, as1_ref[...], preferred_element_type=jnp.float32)
    ald_ref[...] = jnp.dot(h, ad1_ref[...], preferred_element_type=jnp.float32)
    e0, e1, e2, e3 = (eat_ref[0], eat_ref[1], eat_ref[2], eat_ref[3])
    ea1_ref[...] = (e0 * wv_ref[0, 0] + e1 * wv_ref[1, 0]
                    + e2 * wv_ref[2, 0] + e3 * wv_ref[3, 0])
    ea2_ref[...] = (e0 * wv_ref[0, 1] + e1 * wv_ref[1, 1]
                    + e2 * wv_ref[2, 1] + e3 * wv_ref[3, 1])


_tc_prep = pl.pallas_call(
    _tc_prep_body,
    in_specs=[pl.BlockSpec(), pl.BlockSpec(), pl.BlockSpec(), pl.BlockSpec(),
              pl.BlockSpec(), pl.BlockSpec(memory_space=pltpu.SMEM)],
    out_shape=[jax.ShapeDtypeStruct((_NP, _HID), jnp.float32),
               jax.ShapeDtypeStruct((_NP,), jnp.float32),
               jax.ShapeDtypeStruct((_NP,), jnp.float32),
               jax.ShapeDtypeStruct((_EP,), jnp.float32),
               jax.ShapeDtypeStruct((_EP,), jnp.float32)],
)


def _tc_comb1_body(amaxp_ref, degp_ref, esump_ref, als_ref, ald_ref, wv_ref,
                   amax_ref, exl_ref, la2_ref):
    deg = jnp.maximum(degp_ref[0] + degp_ref[1], 1.0)
    es = [esump_ref[0, cc] + esump_ref[1, cc] for cc in range(4)]
    la1 = (es[0] * wv_ref[0, 0] + es[1] * wv_ref[1, 0]
           + es[2] * wv_ref[2, 0] + es[3] * wv_ref[3, 0]) / deg
    la2 = (es[0] * wv_ref[0, 1] + es[1] * wv_ref[1, 1]
           + es[2] * wv_ref[2, 1] + es[3] * wv_ref[3, 1]) / deg
    a_loop = _leaky(als_ref[...] + ald_ref[...] + la1)
    amax = jnp.maximum(jnp.max(amaxp_ref[...], axis=0), a_loop)
    amax_ref[...] = amax
    exl_ref[...] = jnp.exp(a_loop - amax)
    la2_ref[...] = la2


_tc_comb1 = pl.pallas_call(
    _tc_comb1_body,
    in_specs=[pl.BlockSpec(), pl.BlockSpec(), pl.BlockSpec(), pl.BlockSpec(),
              pl.BlockSpec(), pl.BlockSpec(memory_space=pltpu.SMEM)],
    out_shape=[jax.ShapeDtypeStruct((_NP,), jnp.float32),
               jax.ShapeDtypeStruct((_NP,), jnp.float32),
               jax.ShapeDtypeStruct((_NP,), jnp.float32)],
)


def _tc_mid_body(accp_ref, denp_ref, exl_ref, h1_ref, b1_ref, w2_ref,
                 as2_ref, ad2_ref, la2_ref,
                 h2_ref, als2_ref, ald2_ref, aloop2_ref):
    exl = exl_ref[...]
    den = jnp.maximum(denp_ref[0] + denp_ref[1] + exl, 1e-16)
    acc = accp_ref[0] + accp_ref[1] + exl[:, None] * h1_ref[...]
    out1 = acc / den[:, None] + b1_ref[...]
    h1r = jnp.maximum(out1, 0.0)
    h2 = jnp.dot(h1r, w2_ref[...], preferred_element_type=jnp.float32)
    h2_ref[...] = h2
    als2 = jnp.dot(h2, as2_ref[...], preferred_element_type=jnp.float32)
    ald2 = jnp.dot(h2, ad2_ref[...], preferred_element_type=jnp.float32)
    als2_ref[...] = als2
    ald2_ref[...] = ald2
    aloop2_ref[...] = _leaky(als2 + ald2 + la2_ref[...])


_tc_mid = pl.pallas_call(
    _tc_mid_body,
    out_shape=[jax.ShapeDtypeStruct((_NP, _HID), jnp.float32),
               jax.ShapeDtypeStruct((_NP,), jnp.float32),
               jax.ShapeDtypeStruct((_NP,), jnp.float32),
               jax.ShapeDtypeStruct((_NP,), jnp.float32)],
)


def _tc_comb2_body(amaxp_ref, aloop2_ref, amax_ref, exl_ref):
    a_loop = aloop2_ref[...]
    amax = jnp.maximum(jnp.max(amaxp_ref[...], axis=0), a_loop)
    amax_ref[...] = amax
    exl_ref[...] = jnp.exp(a_loop - amax)


_tc_comb2 = pl.pallas_call(
    _tc_comb2_body,
    out_shape=[jax.ShapeDtypeStruct((_NP,), jnp.float32),
               jax.ShapeDtypeStruct((_NP,), jnp.float32)],
)


def _tc_final_body(accp_ref, denp_ref, exl_ref, h2_ref, b2_ref, eps_ref,
                   wmu_ref, bmu_ref, wlv_ref, blv_ref, wd_ref, bd_ref,
                   wa1_ref, ba1_ref, wa2_ref, ba2_ref, wn1_ref, bn1_ref,
                   wn2_ref, bn2_ref,
                   mu_ref, lv_ref, rowa_ref, rown_ref):
    exl = exl_ref[...]
    den = jnp.maximum(denp_ref[0] + denp_ref[1] + exl, 1e-16)
    acc = accp_ref[0] + accp_ref[1] + exl[:, None] * h2_ref[...]
    out2 = acc / den[:, None] + b2_ref[...]
    h2r = jnp.maximum(out2[:_N], 0.0)
    g = jnp.mean(h2r, axis=0, keepdims=True)
    mu = jnp.dot(g, wmu_ref[...], preferred_element_type=jnp.float32) + bmu_ref[...]
    lv = jnp.dot(g, wlv_ref[...], preferred_element_type=jnp.float32) + blv_ref[...]
    mu_ref[...] = mu
    lv_ref[...] = lv
    z = mu + eps_ref[...] * jnp.exp(0.5 * lv)
    hd = jnp.maximum(jnp.dot(z, wd_ref[...], preferred_element_type=jnp.float32) + bd_ref[...], 0.0)
    ha = jnp.maximum(jnp.dot(hd, wa1_ref[...], preferred_element_type=jnp.float32) + ba1_ref[...], 0.0)
    rowa_ref[...] = jnp.dot(ha, wa2_ref[...], preferred_element_type=jnp.float32) + ba2_ref[...]
    hn = jnp.maximum(jnp.dot(hd, wn1_ref[...], preferred_element_type=jnp.float32) + bn1_ref[...], 0.0)
    rown_ref[...] = jnp.dot(hn, wn2_ref[...], preferred_element_type=jnp.float32) + bn2_ref[...]


_tc_final = pl.pallas_call(
    _tc_final_body,
    out_shape=[jax.ShapeDtypeStruct((1, _LD), jnp.float32),
               jax.ShapeDtypeStruct((1, _LD), jnp.float32),
               jax.ShapeDtypeStruct((1, _MAXN), jnp.float32),
               jax.ShapeDtypeStruct((1, _DF), jnp.float32)],
)


def _tc_bcast_body(rowa_ref, rown_ref, adj_ref, nf_ref):
    adj_ref[...] = jnp.broadcast_to(rowa_ref[...][:, None, :], (1, _N, _MAXN))
    nf_ref[...] = jnp.broadcast_to(rown_ref[...][:, None, :], (1, _N, _DF))


_tc_bcast = pl.pallas_call(
    _tc_bcast_body,
    out_shape=[jax.ShapeDtypeStruct((1, _N, _MAXN), jnp.float32),
               jax.ShapeDtypeStruct((1, _N, _DF), jnp.float32)],
)


def _impl(x, edge_index, edge_attr, W1, as1, ad1, ae1, We1, b1, W2, as2, ad2,
          ae2, We2, b2, Wmu, bmu, Wlv, blv, Wd, bd, Wa1, ba1, Wa2, ba2, Wn1,
          bn1, Wn2, bn2):
    npad = _EP - _E
    pad_idx = (_N + jnp.arange(npad, dtype=jnp.int32) % 128)
    srcp = jnp.concatenate([edge_index[0], pad_idx]).reshape(_RP, _L)
    dstp = jnp.concatenate([edge_index[1], pad_idx]).reshape(_RP, _L)
    eat2 = jnp.concatenate(
        [edge_attr.T, jnp.zeros((_DE, npad), jnp.float32)], axis=1)
    eat3 = eat2.reshape(_DE, _RP, _L)
    xp = jnp.concatenate([x, jnp.zeros((_NP - _N, _DF), jnp.float32)], axis=0)
    wv = jnp.stack([We1 @ ae1, We2 @ ae2], axis=1)
    eps = jax.random.normal(jax.random.key(42), (1, _LD), dtype=jnp.float32)

    h1, als1, ald1, ea1f, ea2f = _tc_prep(xp, W1, as1, ad1, eat2, wv)
    ea1 = ea1f.reshape(_RP, _L)
    ea2 = ea2f.reshape(_RP, _L)

    amax1p, degp, esump = _make_sc_pass_a(True)(srcp, dstp, ea1, eat3, als1,
                                                ald1)
    amax1, exl1, la2 = _tc_comb1(amax1p.reshape(_NW, _NP),
                                 degp.reshape(2, _NP),
                                 esump.reshape(2, 4, _NP), als1, ald1, wv)

    sc_pass_b = _make_sc_pass_b()
    acc1p, den1p = sc_pass_b(srcp, dstp, ea1, als1, ald1, amax1, h1)
    h2, als2, ald2, aloop2 = _tc_mid(acc1p.reshape(2, _NP, _HID),
                                     den1p.reshape(2, _NP), exl1, h1, b1, W2,
                                     as2, ad2, la2)

    amax2p = _make_sc_pass_a(False)(srcp, dstp, ea2, als2, ald2)
    if isinstance(amax2p, (list, tuple)):
        amax2p = amax2p[0]
    amax2, exl2 = _tc_comb2(amax2p.reshape(_NW, _NP), aloop2)

    acc2p, den2p = sc_pass_b(srcp, dstp, ea2, als2, ald2, amax2, h2)
    mu, logvar, rowa, rown = _tc_final(
        acc2p.reshape(2, _NP, _HID), den2p.reshape(2, _NP), exl2, h2, b2, eps,
        Wmu, bmu, Wlv, blv, Wd, bd, Wa1, ba1, Wa2, ba2, Wn1, bn1, Wn2, bn2)

    adj, nf = _tc_bcast(rowa, rown)
    return adj, nf, mu, logvar


def kernel(*args):
    return _impl(*args)


# speedup vs baseline: 32.1159x; 1.2643x over previous
import functools

import jax
import jax.numpy as jnp
from jax import lax
from jax.experimental import pallas as pl
from jax.experimental.pallas import tpu as pltpu
from jax.experimental.pallas import tpu_sc as plsc

_N = 10000
_E = 320000
_DF = 128
_DE = 4
_HID = 64
_LD = 32
_MAXN = 200

_NP = 10240
_NPT = _NP // 16
_L = 128
_NW = 32
_CROWS = 16
_NCH = 5
_RPW = _CROWS * _NCH
_RP = _NW * _RPW
_EP = _RP * _L


@functools.lru_cache(maxsize=None)
def _sc_mesh():
    return plsc.VectorSubcoreMesh(core_axis_name="c", subcore_axis_name="s")


def _leaky(v):
    return jnp.maximum(v, 0.2 * v)


def _edge_logit_vreg(als_v, ald_v, ea_c, src_c, dst_c, j, l):
    sl = pl.ds(l * 16, 16)
    s16 = src_c[j, sl]
    d16 = dst_c[j, sl]
    a = (plsc.load_gather(als_v, [s16]) + plsc.load_gather(ald_v, [d16])
         + ea_c[j, sl])
    return _leaky(a), d16


def _scatter_max_vreg(amax_v, d16, a):
    def cond(done):
        return done == 0

    def body(done):
        cur = plsc.load_gather(amax_v, [d16])
        plsc.store_scatter(amax_v, [d16], a, mask=a > cur)
        chk = plsc.load_gather(amax_v, [d16])
        return lax.select(jnp.all(chk >= a), 1, 0)

    lax.while_loop(cond, body, 0)


def _zero_vec(ref, n):
    def zbody(i, _):
        ref[pl.ds(i * 16, 16)] = jnp.zeros((16,), jnp.float32)
        return 0
    lax.fori_loop(0, n // 16, zbody, 0)


def _sc_pass_a_body(with_esum, src_hbm, dst_hbm, ea_hbm, eat_hbm, als_hbm,
                    ald_hbm, amax_out, deg_out, esum_out,
                    als_v, ald_v, amax_v, src_c, dst_c, ea_c, eat_c, ones_v,
                    bnc_v, deg_sh, e0_sh, e1_sh, e2_sh, e3_sh):
    c = lax.axis_index("c")
    s = lax.axis_index("s")
    wid = s * 2 + c
    tsl = pl.ds(s * _NPT, _NPT)

    if with_esum:
        _zero_vec(bnc_v, _NPT)
        pltpu.sync_copy(bnc_v, deg_sh.at[tsl])
        pltpu.sync_copy(bnc_v, e0_sh.at[tsl])
        pltpu.sync_copy(bnc_v, e1_sh.at[tsl])
        pltpu.sync_copy(bnc_v, e2_sh.at[tsl])
        pltpu.sync_copy(bnc_v, e3_sh.at[tsl])

    pltpu.sync_copy(als_hbm, als_v)
    pltpu.sync_copy(ald_hbm, ald_v)

    def init_amax(i, _):
        amax_v[pl.ds(i * 16, 16)] = jnp.full((16,), -1e38, jnp.float32)
        return 0
    lax.fori_loop(0, _NP // 16, init_amax, 0)

    if with_esum:
        def init_ones(i, _):
            ones_v[pl.ds(i * 16, 16)] = jnp.full((16,), 1.0, jnp.float32)
            return 0
        lax.fori_loop(0, _L // 16, init_ones, 0)
        plsc.subcore_barrier()

    def do_row(j, _):
        if with_esum:
            idx = dst_c.at[j]
            pltpu.sync_copy(ones_v, deg_sh.at[idx], add=True)
            pltpu.sync_copy(eat_c.at[0, j], e0_sh.at[idx], add=True)
            pltpu.sync_copy(eat_c.at[1, j], e1_sh.at[idx], add=True)
            pltpu.sync_copy(eat_c.at[2, j], e2_sh.at[idx], add=True)
            pltpu.sync_copy(eat_c.at[3, j], e3_sh.at[idx], add=True)

        def vbody(l, _):
            a, d16 = _edge_logit_vreg(als_v, ald_v, ea_c, src_c, dst_c, j, l)
            _scatter_max_vreg(amax_v, d16, a)
            return 0
        lax.fori_loop(0, _L // 16, vbody, 0)
        return 0

    def do_chunk(ch, _):
        rowbase = wid * _RPW + ch * _CROWS
        rsl = pl.ds(rowbase, _CROWS)
        pltpu.sync_copy(src_hbm.at[rsl], src_c)
        pltpu.sync_copy(dst_hbm.at[rsl], dst_c)
        pltpu.sync_copy(ea_hbm.at[rsl], ea_c)
        if with_esum:
            for cc in range(4):
                pltpu.sync_copy(eat_hbm.at[cc, rsl], eat_c.at[cc])
        lax.fori_loop(0, _CROWS, do_row, 0)
        return 0

    lax.fori_loop(0, _NCH, do_chunk, 0)

    pltpu.sync_copy(amax_v, amax_out.at[pl.ds(wid * _NP, _NP)])

    if with_esum:
        plsc.subcore_barrier()
        base = c * _NP + s * _NPT
        osl = pl.ds(base, _NPT)
        pltpu.sync_copy(deg_sh.at[tsl], bnc_v)
        pltpu.sync_copy(bnc_v, deg_out.at[osl])
        for i, ref in enumerate((e0_sh, e1_sh, e2_sh, e3_sh)):
            pltpu.sync_copy(ref.at[tsl], bnc_v)
            pltpu.sync_copy(
                bnc_v, esum_out.at[pl.ds(c * 4 * _NP + i * _NP + s * _NPT,
                                         _NPT)])


@functools.lru_cache(maxsize=None)
def _make_sc_pass_a(with_esum):
    outs = [jax.ShapeDtypeStruct((_NW * _NP,), jnp.float32)]
    if with_esum:
        outs += [jax.ShapeDtypeStruct((2 * _NP,), jnp.float32),
                 jax.ShapeDtypeStruct((2 * 4 * _NP,), jnp.float32)]
    scratch = [
        pltpu.VMEM((_NP,), jnp.float32),
        pltpu.VMEM((_NP,), jnp.float32),
        pltpu.VMEM((_NP,), jnp.float32),
        pltpu.VMEM((_CROWS, _L), jnp.int32),
        pltpu.VMEM((_CROWS, _L), jnp.int32),
        pltpu.VMEM((_CROWS, _L), jnp.float32),
        pltpu.VMEM((4, _CROWS, _L), jnp.float32),
        pltpu.VMEM((_L,), jnp.float32),
        pltpu.VMEM((_NPT,), jnp.float32),
        pltpu.VMEM_SHARED((_NP,), jnp.float32),
        pltpu.VMEM_SHARED((_NP,), jnp.float32),
        pltpu.VMEM_SHARED((_NP,), jnp.float32),
        pltpu.VMEM_SHARED((_NP,), jnp.float32),
        pltpu.VMEM_SHARED((_NP,), jnp.float32),
    ]

    if with_esum:
        def body(src_hbm, dst_hbm, ea_hbm, eat_hbm, als_hbm, ald_hbm,
                 amax_out, deg_out, esum_out, *sc):
            _sc_pass_a_body(True, src_hbm, dst_hbm, ea_hbm, eat_hbm, als_hbm,
                            ald_hbm, amax_out, deg_out, esum_out, *sc)
    else:
        def body(src_hbm, dst_hbm, ea_hbm, als_hbm, ald_hbm, amax_out, *sc):
            _sc_pass_a_body(False, src_hbm, dst_hbm, ea_hbm, None, als_hbm,
                            ald_hbm, amax_out, None, None, *sc)

    return pl.kernel(body, mesh=_sc_mesh(), out_type=outs,
                     scratch_types=scratch,
                     compiler_params=pltpu.CompilerParams(
                         needs_layout_passes=False,
                         use_tc_tiling_on_sc=False))


_NBUF = 4


def _sc_pass_b_body(src_hbm, dst_hbm, ea_hbm, als_hbm, ald_hbm, amax_hbm,
                    h_hbm, acc_out, den_out,
                    als_v, ald_v, amax_v, src_c, dst_c, ea_c, exs_c,
                    rows0, rows1, rows2, rows3,
                    bnc_v, acc_sh, den_sh,
                    gsem0, gsem1, gsem2, gsem3,
                    ssem0, ssem1, ssem2, ssem3, dsem):
    c = lax.axis_index("c")
    s = lax.axis_index("s")
    wid = s * 2 + c
    tsl = pl.ds(s * _NPT, _NPT)
    bufs = (rows0, rows1, rows2, rows3)
    gsems = (gsem0, gsem1, gsem2, gsem3)
    ssems = (ssem0, ssem1, ssem2, ssem3)

    _zero_vec(bnc_v, _NPT)
    pltpu.sync_copy(bnc_v, den_sh.at[tsl])

    def zrow(r, _):
        for cb in range(4):
            rows0[r, pl.ds(cb * 16, 16)] = jnp.zeros((16,), jnp.float32)
        return 0
    lax.fori_loop(0, _L, zrow, 0)
    for k in range(_NPT // _L):
        pltpu.sync_copy(rows0, acc_sh.at[pl.ds(s * _NPT + k * _L, _L)])

    pltpu.sync_copy(als_hbm, als_v)
    pltpu.sync_copy(ald_hbm, ald_v)
    pltpu.sync_copy(amax_hbm, amax_v)
    plsc.subcore_barrier()

    def do_chunk(ch, _):
        rowbase = wid * _RPW + ch * _CROWS
        rsl = pl.ds(rowbase, _CROWS)
        pltpu.sync_copy(src_hbm.at[rsl], src_c)
        pltpu.sync_copy(dst_hbm.at[rsl], dst_c)
        pltpu.sync_copy(ea_hbm.at[rsl], ea_c)

        def vbody(i, _):
            j = i // (_L // 16)
            l = i % (_L // 16)
            a, d16 = _edge_logit_vreg(als_v, ald_v, ea_c, src_c, dst_c, j, l)
            ex = jnp.exp(a - plsc.load_gather(amax_v, [d16]))
            exs_c[j, pl.ds(l * 16, 16)] = ex
            return 0
        lax.fori_loop(0, _CROWS * (_L // 16), vbody, 0)

        gh = [None] * _NBUF
        sh = [None] * _NBUF
        dh = []
        for j in range(_NBUF - 1):
            b = j % _NBUF
            gh[b] = pltpu.async_copy(h_hbm.at[src_c.at[j]], bufs[b], gsems[b])
        for j in range(_CROWS):
            b = j % _NBUF
            gh[b].wait()
            buf = bufs[b]

            def sbody(l, _, j=j, buf=buf):
                ex16 = exs_c[j, pl.ds(l * 16, 16)]
                for i in range(16):
                    e = ex16[i]
                    r = l * 16 + i
                    for cb in range(4):
                        sl = pl.ds(cb * 16, 16)
                        buf[r, sl] = buf[r, sl] * e
                return 0
            lax.fori_loop(0, _L // 16, sbody, 0)
            idx = dst_c.at[j]
            sh[b] = pltpu.async_copy(buf, acc_sh.at[idx], ssems[b], add=True)
            dh.append(
                pltpu.async_copy(exs_c.at[j], den_sh.at[idx], dsem, add=True))
            nxt = j + _NBUF - 1
            if nxt < _CROWS:
                nb = nxt % _NBUF
                if sh[nb] is not None:
                    sh[nb].wait()
                gh[nb] = pltpu.async_copy(h_hbm.at[src_c.at[nxt]], bufs[nb],
                                          gsems[nb])
        for b in range(_NBUF):
            if sh[b] is not None:
                sh[b].wait()
        for h in dh:
            h.wait()
        return 0

    lax.fori_loop(0, _NCH, do_chunk, 0)

    plsc.subcore_barrier()

    pltpu.sync_copy(den_sh.at[tsl], bnc_v)
    pltpu.sync_copy(bnc_v, den_out.at[pl.ds(c * _NP + s * _NPT, _NPT)])
    for k in range(_NPT // _L):
        ssl = pl.ds(s * _NPT + k * _L, _L)
        pltpu.sync_copy(acc_sh.at[ssl], rows0)
        pltpu.sync_copy(rows0,
                        acc_out.at[pl.ds(c * _NP + s * _NPT + k * _L, _L)])


@functools.lru_cache(maxsize=None)
def _make_sc_pass_b():
    return pl.kernel(
        _sc_pass_b_body,
        mesh=_sc_mesh(),
        out_type=[jax.ShapeDtypeStruct((2 * _NP, _HID), jnp.float32),
                  jax.ShapeDtypeStruct((2 * _NP,), jnp.float32)],
        scratch_types=[
            pltpu.VMEM((_NP,), jnp.float32),
            pltpu.VMEM((_NP,), jnp.float32),
            pltpu.VMEM((_NP,), jnp.float32),
            pltpu.VMEM((_CROWS, _L), jnp.int32),
            pltpu.VMEM((_CROWS, _L), jnp.int32),
            pltpu.VMEM((_CROWS, _L), jnp.float32),
            pltpu.VMEM((_CROWS, _L), jnp.float32),
            pltpu.VMEM((_L, _HID), jnp.float32),
            pltpu.VMEM((_L, _HID), jnp.float32),
            pltpu.VMEM((_L, _HID), jnp.float32),
            pltpu.VMEM((_L, _HID), jnp.float32),
            pltpu.VMEM((_NPT,), jnp.float32),
            pltpu.VMEM_SHARED((_NP, _HID), jnp.float32),
            pltpu.VMEM_SHARED((_NP,), jnp.float32),
            pltpu.SemaphoreType.DMA,
            pltpu.SemaphoreType.DMA,
            pltpu.SemaphoreType.DMA,
            pltpu.SemaphoreType.DMA,
            pltpu.SemaphoreType.DMA,
            pltpu.SemaphoreType.DMA,
            pltpu.SemaphoreType.DMA,
            pltpu.SemaphoreType.DMA,
            pltpu.SemaphoreType.DMA,
        ],
        compiler_params=pltpu.CompilerParams(needs_layout_passes=False,
                                             use_tc_tiling_on_sc=False),
    )


def _tc_prep_body(x_ref, w1_ref, as1_ref, ad1_ref, eat_ref, wv_ref,
                  h1_ref, als_ref, ald_ref, ea1_ref, ea2_ref):
    h = jnp.dot(x_ref[...], w1_ref[...], preferred_element_type=jnp.float32)
    h1_ref[...] = h
    als_ref[...] = jnp.dot(h, as1_ref[...], preferred_element_type=jnp.float32)
    ald_ref[...] = jnp.dot(h, ad1_ref[...], preferred_element_type=jnp.float32)
    e0, e1, e2, e3 = (eat_ref[0], eat_ref[1], eat_ref[2], eat_ref[3])
    ea1_ref[...] = (e0 * wv_ref[0, 0] + e1 * wv_ref[1, 0]
                    + e2 * wv_ref[2, 0] + e3 * wv_ref[3, 0])
    ea2_ref[...] = (e0 * wv_ref[0, 1] + e1 * wv_ref[1, 1]
                    + e2 * wv_ref[2, 1] + e3 * wv_ref[3, 1])


_tc_prep = pl.pallas_call(
    _tc_prep_body,
    in_specs=[pl.BlockSpec(), pl.BlockSpec(), pl.BlockSpec(), pl.BlockSpec(),
              pl.BlockSpec(), pl.BlockSpec(memory_space=pltpu.SMEM)],
    out_shape=[jax.ShapeDtypeStruct((_NP, _HID), jnp.float32),
               jax.ShapeDtypeStruct((_NP,), jnp.float32),
               jax.ShapeDtypeStruct((_NP,), jnp.float32),
               jax.ShapeDtypeStruct((_EP,), jnp.float32),
               jax.ShapeDtypeStruct((_EP,), jnp.float32)],
)


def _tc_comb1_body(amaxp_ref, degp_ref, esump_ref, als_ref, ald_ref, wv_ref,
                   amax_ref, exl_ref, la2_ref):
    deg = jnp.maximum(degp_ref[0] + degp_ref[1], 1.0)
    es = [esump_ref[0, cc] + esump_ref[1, cc] for cc in range(4)]
    la1 = (es[0] * wv_ref[0, 0] + es[1] * wv_ref[1, 0]
           + es[2] * wv_ref[2, 0] + es[3] * wv_ref[3, 0]) / deg
    la2 = (es[0] * wv_ref[0, 1] + es[1] * wv_ref[1, 1]
           + es[2] * wv_ref[2, 1] + es[3] * wv_ref[3, 1]) / deg
    a_loop = _leaky(als_ref[...] + ald_ref[...] + la1)
    amax = jnp.maximum(jnp.max(amaxp_ref[...], axis=0), a_loop)
    amax_ref[...] = amax
    exl_ref[...] = jnp.exp(a_loop - amax)
    la2_ref[...] = la2


_tc_comb1 = pl.pallas_call(
    _tc_comb1_body,
    in_specs=[pl.BlockSpec(), pl.BlockSpec(), pl.BlockSpec(), pl.BlockSpec(),
              pl.BlockSpec(), pl.BlockSpec(memory_space=pltpu.SMEM)],
    out_shape=[jax.ShapeDtypeStruct((_NP,), jnp.float32),
               jax.ShapeDtypeStruct((_NP,), jnp.float32),
               jax.ShapeDtypeStruct((_NP,), jnp.float32)],
)


def _tc_mid_body(accp_ref, denp_ref, exl_ref, h1_ref, b1_ref, w2_ref,
                 as2_ref, ad2_ref, la2_ref,
                 h2_ref, als2_ref, ald2_ref, aloop2_ref):
    exl = exl_ref[...]
    den = jnp.maximum(denp_ref[0] + denp_ref[1] + exl, 1e-16)
    acc = accp_ref[0] + accp_ref[1] + exl[:, None] * h1_ref[...]
    out1 = acc / den[:, None] + b1_ref[...]
    h1r = jnp.maximum(out1, 0.0)
    h2 = jnp.dot(h1r, w2_ref[...], preferred_element_type=jnp.float32)
    h2_ref[...] = h2
    als2 = jnp.dot(h2, as2_ref[...], preferred_element_type=jnp.float32)
    ald2 = jnp.dot(h2, ad2_ref[...], preferred_element_type=jnp.float32)
    als2_ref[...] = als2
    ald2_ref[...] = ald2
    aloop2_ref[...] = _leaky(als2 + ald2 + la2_ref[...])


_tc_mid = pl.pallas_call(
    _tc_mid_body,
    out_shape=[jax.ShapeDtypeStruct((_NP, _HID), jnp.float32),
               jax.ShapeDtypeStruct((_NP,), jnp.float32),
               jax.ShapeDtypeStruct((_NP,), jnp.float32),
               jax.ShapeDtypeStruct((_NP,), jnp.float32)],
)


def _tc_comb2_body(amaxp_ref, aloop2_ref, amax_ref, exl_ref):
    a_loop = aloop2_ref[...]
    amax = jnp.maximum(jnp.max(amaxp_ref[...], axis=0), a_loop)
    amax_ref[...] = amax
    exl_ref[...] = jnp.exp(a_loop - amax)


_tc_comb2 = pl.pallas_call(
    _tc_comb2_body,
    out_shape=[jax.ShapeDtypeStruct((_NP,), jnp.float32),
               jax.ShapeDtypeStruct((_NP,), jnp.float32)],
)


def _tc_final_body(accp_ref, denp_ref, exl_ref, h2_ref, b2_ref, eps_ref,
                   wmu_ref, bmu_ref, wlv_ref, blv_ref, wd_ref, bd_ref,
                   wa1_ref, ba1_ref, wa2_ref, ba2_ref, wn1_ref, bn1_ref,
                   wn2_ref, bn2_ref,
                   mu_ref, lv_ref, rowa_ref, rown_ref):
    exl = exl_ref[...]
    den = jnp.maximum(denp_ref[0] + denp_ref[1] + exl, 1e-16)
    acc = accp_ref[0] + accp_ref[1] + exl[:, None] * h2_ref[...]
    out2 = acc / den[:, None] + b2_ref[...]
    h2r = jnp.maximum(out2[:_N], 0.0)
    g = jnp.mean(h2r, axis=0, keepdims=True)
    mu = jnp.dot(g, wmu_ref[...], preferred_element_type=jnp.float32) + bmu_ref[...]
    lv = jnp.dot(g, wlv_ref[...], preferred_element_type=jnp.float32) + blv_ref[...]
    mu_ref[...] = mu
    lv_ref[...] = lv
    z = mu + eps_ref[...] * jnp.exp(0.5 * lv)
    hd = jnp.maximum(jnp.dot(z, wd_ref[...], preferred_element_type=jnp.float32) + bd_ref[...], 0.0)
    ha = jnp.maximum(jnp.dot(hd, wa1_ref[...], preferred_element_type=jnp.float32) + ba1_ref[...], 0.0)
    rowa_ref[...] = jnp.dot(ha, wa2_ref[...], preferred_element_type=jnp.float32) + ba2_ref[...]
    hn = jnp.maximum(jnp.dot(hd, wn1_ref[...], preferred_element_type=jnp.float32) + bn1_ref[...], 0.0)
    rown_ref[...] = jnp.dot(hn, wn2_ref[...], preferred_element_type=jnp.float32) + bn2_ref[...]


_tc_final = pl.pallas_call(
    _tc_final_body,
    out_shape=[jax.ShapeDtypeStruct((1, _LD), jnp.float32),
               jax.ShapeDtypeStruct((1, _LD), jnp.float32),
               jax.ShapeDtypeStruct((1, _MAXN), jnp.float32),
               jax.ShapeDtypeStruct((1, _DF), jnp.float32)],
)


def _tc_bcast_body(rowa_ref, rown_ref, adj_ref, nf_ref):
    adj_ref[...] = jnp.broadcast_to(rowa_ref[...][:, None, :], (1, _N, _MAXN))
    nf_ref[...] = jnp.broadcast_to(rown_ref[...][:, None, :], (1, _N, _DF))


_tc_bcast = pl.pallas_call(
    _tc_bcast_body,
    out_shape=[jax.ShapeDtypeStruct((1, _N, _MAXN), jnp.float32),
               jax.ShapeDtypeStruct((1, _N, _DF), jnp.float32)],
)


def _impl(x, edge_index, edge_attr, W1, as1, ad1, ae1, We1, b1, W2, as2, ad2,
          ae2, We2, b2, Wmu, bmu, Wlv, blv, Wd, bd, Wa1, ba1, Wa2, ba2, Wn1,
          bn1, Wn2, bn2):
    npad = _EP - _E
    pad_idx = (_N + jnp.arange(npad, dtype=jnp.int32) % 128)
    srcp = jnp.concatenate([edge_index[0], pad_idx]).reshape(_RP, _L)
    dstp = jnp.concatenate([edge_index[1], pad_idx]).reshape(_RP, _L)
    eat2 = jnp.concatenate(
        [edge_attr.T, jnp.zeros((_DE, npad), jnp.float32)], axis=1)
    eat3 = eat2.reshape(_DE, _RP, _L)
    xp = jnp.concatenate([x, jnp.zeros((_NP - _N, _DF), jnp.float32)], axis=0)
    wv = jnp.stack([We1 @ ae1, We2 @ ae2], axis=1)
    eps = jax.random.normal(jax.random.key(42), (1, _LD), dtype=jnp.float32)

    h1, als1, ald1, ea1f, ea2f = _tc_prep(xp, W1, as1, ad1, eat2, wv)
    ea1 = ea1f.reshape(_RP, _L)
    ea2 = ea2f.reshape(_RP, _L)

    amax1p, degp, esump = _make_sc_pass_a(True)(srcp, dstp, ea1, eat3, als1,
                                                ald1)
    amax1, exl1, la2 = _tc_comb1(amax1p.reshape(_NW, _NP),
                                 degp.reshape(2, _NP),
                                 esump.reshape(2, 4, _NP), als1, ald1, wv)

    sc_pass_b = _make_sc_pass_b()
    acc1p, den1p = sc_pass_b(srcp, dstp, ea1, als1, ald1, amax1, h1)
    h2, als2, ald2, aloop2 = _tc_mid(acc1p.reshape(2, _NP, _HID),
                                     den1p.reshape(2, _NP), exl1, h1, b1, W2,
                                     as2, ad2, la2)

    amax2p = _make_sc_pass_a(False)(srcp, dstp, ea2, als2, ald2)
    if isinstance(amax2p, (list, tuple)):
        amax2p = amax2p[0]
    amax2, exl2 = _tc_comb2(amax2p.reshape(_NW, _NP), aloop2)

    acc2p, den2p = sc_pass_b(srcp, dstp, ea2, als2, ald2, amax2, h2)
    mu, logvar, rowa, rown = _tc_final(
        acc2p.reshape(2, _NP, _HID), den2p.reshape(2, _NP), exl2, h2, b2, eps,
        Wmu, bmu, Wlv, blv, Wd, bd, Wa1, ba1, Wa2, ba2, Wn1, bn1, Wn2, bn2)

    adj, nf = _tc_bcast(rowa, rown)
    return adj, nf, mu, logvar


def kernel(*args):
    return _impl(*args)
